# baseline (speedup 1.0000x reference)
import functools

import jax
import jax.numpy as jnp
from jax import lax
from jax.experimental import pallas as pl
from jax.experimental.pallas import tpu as pltpu
from jax.experimental.pallas import tpu_sc as plsc

M_TOK = 2048
D_IN = 768
D_OUT = 768
N_EXP = 64
LANES = 16
N_SUB = 16
TOK_PER_SUB = M_TOK // N_SUB
ROW_I32 = D_IN // 2
TM = 64
OFFS_PAD = 80


def _sc_sort_body(eidx_hbm, x_hbm, xs_hbm, offs_hbm,
                  shared_hist, shared_sid, ids_v, hist_v, cnts_v, base_v,
                  offs_v, pos_v, tok_v, sid_v, rows_v, sem):
  wid = lax.axis_index("s")
  tbase = wid * TOK_PER_SUB
  zeros = jnp.zeros((LANES,), jnp.int32)

  pltpu.sync_copy(eidx_hbm.at[pl.ds(tbase, TOK_PER_SUB)], ids_v)

  for k in range(N_EXP // LANES):
    hist_v[pl.ds(k * LANES, LANES)] = zeros
  for k in range(TOK_PER_SUB // LANES):
    idvec = ids_v[pl.ds(k * LANES, LANES)]
    occ, last = plsc.scan_count(idvec)
    prior = plsc.load_gather(hist_v, [idvec])
    plsc.store_scatter(hist_v, [idvec], prior + occ, mask=last)

  pltpu.sync_copy(hist_v, shared_hist.at[wid])
  plsc.subcore_barrier()
  pltpu.sync_copy(shared_hist, cnts_v)

  carry = jnp.int32(0)
  for k in range(N_EXP // LANES):
    tot = zeros
    pre = zeros
    for w2 in range(N_SUB):
      row = cnts_v[w2, pl.ds(k * LANES, LANES)]
      tot = tot + row
      pre = pre + jnp.where(w2 < wid, row, zeros)
    incl = plsc.cumsum(tot)
    excl = incl - tot + carry
    base_v[pl.ds(k * LANES, LANES)] = excl + pre
    offs_v[pl.ds(k * LANES, LANES)] = excl
    carry = carry + jnp.sum(tot)
  lane = lax.iota(jnp.int32, LANES)
  offs_v[pl.ds(N_EXP, LANES)] = jnp.where(lane == 0, jnp.int32(M_TOK), 0)

  for k in range(N_EXP // LANES):
    hist_v[pl.ds(k * LANES, LANES)] = zeros
  for k in range(TOK_PER_SUB // LANES):
    idvec = ids_v[pl.ds(k * LANES, LANES)]
    occ, last = plsc.scan_count(idvec)
    prior = plsc.load_gather(hist_v, [idvec])
    basee = plsc.load_gather(base_v, [idvec])
    pos_v[0, pl.ds(k * LANES, LANES)] = basee + prior + occ - 1
    tok_v[pl.ds(k * LANES, LANES)] = tbase + k * LANES + lane
    plsc.store_scatter(hist_v, [idvec], prior + occ, mask=last)

  pltpu.sync_copy(tok_v, shared_sid.at[pos_v.at[0]])
  plsc.subcore_barrier()
  pltpu.sync_copy(shared_sid.at[pl.ds(tbase, TOK_PER_SUB)], sid_v)
  pltpu.async_copy(x_hbm.at[sid_v], rows_v, sem).wait()
  pltpu.sync_copy(rows_v, xs_hbm.at[pl.ds(tbase, TOK_PER_SUB)])

  @pl.when(wid == 0)
  def _():
    pltpu.sync_copy(offs_v, offs_hbm)


@functools.cache
def _get_sc_sort():
  return pl.kernel(
      _sc_sort_body,
      out_type=(
          jax.ShapeDtypeStruct((M_TOK, ROW_I32), jnp.int32),
          jax.ShapeDtypeStruct((OFFS_PAD,), jnp.int32),
      ),
      mesh=plsc.VectorSubcoreMesh(
          core_axis_name="c", subcore_axis_name="s", num_cores=1,
          num_subcores=N_SUB),
      compiler_params=pltpu.CompilerParams(needs_layout_passes=False),
      scratch_types=[
          pltpu.VMEM_SHARED((N_SUB, N_EXP), jnp.int32),
          pltpu.VMEM_SHARED((M_TOK,), jnp.int32),
          pltpu.VMEM((TOK_PER_SUB,), jnp.int32),
          pltpu.VMEM((N_EXP,), jnp.int32),
          pltpu.VMEM((N_SUB, N_EXP), jnp.int32),
          pltpu.VMEM((N_EXP,), jnp.int32),
          pltpu.VMEM((OFFS_PAD,), jnp.int32),
          pltpu.VMEM((1, TOK_PER_SUB), jnp.int32),
          pltpu.VMEM((TOK_PER_SUB,), jnp.int32),
          pltpu.VMEM((TOK_PER_SUB,), jnp.int32),
          pltpu.VMEM((TOK_PER_SUB, ROW_I32), jnp.int32),
          pltpu.SemaphoreType.DMA,
      ],
  )


N_TILES = M_TOK // TM


def _dec_f16(u):
  f32b = ((u & 0x8000) << 16) | (((u & 0x7FFF) << 13) + (112 << 23))
  return lax.bitcast_convert_type(f32b, jnp.float32)


HALF_D = D_IN // 2


def _tc_matmul_body(offs_ref, x_ref, w_ref, out_ref, acc_ref, wp_ref):
  e = pl.program_id(0)

  @pl.when(e == 0)
  def _():
    acc_ref[...] = jnp.zeros_like(acc_ref)

  off0 = offs_ref[e]
  off1 = offs_ref[e + 1]
  t0 = off0 // TM
  t1 = lax.div(off1 + TM - 1, TM)

  w32 = w_ref.bitcast(jnp.int32)[0]
  t = (((((w32 & 0x7FFC7FFC) + 0x00040004)
         >> 3) & 0x1FFF1FFF)
       + 0x38003800)
  wp_ref[...] = t | (w32 & jnp.int32(-2147450880))
  wv = wp_ref.bitcast(jnp.bfloat16)
  we = wv[pl.ds(0, HALF_D), :]
  wo = wv[pl.ds(HALF_D, HALF_D), :]

  def tile_step(k, carry):
    tt = t0 + k
    v = x_ref[tt]
    ridx = tt * TM + lax.broadcasted_iota(jnp.int32, (TM, 1), 0)
    m = (ridx >= off0) & (ridx < off1)
    xlo = jnp.where(m, _dec_f16(v & 0xFFFF), 0.).astype(jnp.bfloat16)
    xhi = jnp.where(m, _dec_f16((v >> 16) & 0xFFFF), 0.).astype(jnp.bfloat16)
    acc_ref[tt] = (acc_ref[tt]
                   + jnp.dot(xlo, we, preferred_element_type=jnp.float32)
                   + jnp.dot(xhi, wo, preferred_element_type=jnp.float32))
    return carry
  lax.fori_loop(0, t1 - t0, tile_step, 0)

  @pl.when(e == N_EXP - 1)
  def _():
    b = lax.bitcast_convert_type(acc_ref[...], jnp.int32)
    sign = (b >> 16) & 0x8000
    t = ((b & 0x7FFFFFFF) + 0x1000) >> 13
    h = jnp.where(t < (112 << 10), 0, t - (112 << 10))
    out_ref.bitcast(jnp.int16)[...] = (sign | h).astype(jnp.int16)


_tc_matmul = pl.pallas_call(
    _tc_matmul_body,
    grid_spec=pltpu.PrefetchScalarGridSpec(
        num_scalar_prefetch=1,
        grid=(N_EXP,),
        in_specs=[
            pl.BlockSpec((N_TILES, TM, ROW_I32), lambda e, s: (0, 0, 0)),
            pl.BlockSpec((1, D_IN, D_OUT), lambda e, s: (e, 0, 0)),
        ],
        out_specs=pl.BlockSpec((N_TILES, TM, D_OUT), lambda e, s: (0, 0, 0)),
        scratch_shapes=[pltpu.VMEM((N_TILES, TM, D_OUT), jnp.float32),
                        pltpu.VMEM((HALF_D, D_OUT), jnp.int32)],
    ),
    out_shape=jax.ShapeDtypeStruct((N_TILES, TM, D_OUT), jnp.float16),
    compiler_params=pltpu.CompilerParams(
        dimension_semantics=("arbitrary",)),
)


def kernel(x, expert_indices, weights):
  x_i32 = lax.bitcast_convert_type(
      jnp.stack([x[:, :HALF_D], x[:, HALF_D:]], axis=2), jnp.int32)
  xs_i32, offs = _get_sc_sort()(expert_indices, x_i32)
  w_bits = lax.bitcast_convert_type(weights, jnp.int16)
  out = _tc_matmul(offs, xs_i32.reshape(N_TILES, TM, ROW_I32), w_bits)
  return out.reshape(M_TOK, D_OUT)


# speedup vs baseline: 1.1149x; 1.1149x over previous
import functools

import jax
import jax.numpy as jnp
from jax import lax
from jax.experimental import pallas as pl
from jax.experimental.pallas import tpu as pltpu
from jax.experimental.pallas import tpu_sc as plsc

M_TOK = 2048
D_IN = 768
D_OUT = 768
N_EXP = 64
LANES = 16
N_SUB = 16
TOK_PER_SUB = M_TOK // N_SUB
ROW_I32 = D_IN // 2
TM = 64
OFFS_PAD = 80


def _sc_sort_body(eidx_hbm, x_hbm, xs_hbm, offs_hbm,
                  shared_hist, shared_sid, ids_v, hist_v, cnts_v, base_v,
                  offs_v, pos_v, tok_v, sid_v, rows_v, sem):
  wid = lax.axis_index("s")
  tbase = wid * TOK_PER_SUB
  zeros = jnp.zeros((LANES,), jnp.int32)

  pltpu.sync_copy(eidx_hbm.at[pl.ds(tbase, TOK_PER_SUB)], ids_v)

  for k in range(N_EXP // LANES):
    hist_v[pl.ds(k * LANES, LANES)] = zeros
  for k in range(TOK_PER_SUB // LANES):
    idvec = ids_v[pl.ds(k * LANES, LANES)]
    occ, last = plsc.scan_count(idvec)
    prior = plsc.load_gather(hist_v, [idvec])
    plsc.store_scatter(hist_v, [idvec], prior + occ, mask=last)

  pltpu.sync_copy(hist_v, shared_hist.at[wid])
  plsc.subcore_barrier()
  pltpu.sync_copy(shared_hist, cnts_v)

  carry = jnp.int32(0)
  for k in range(N_EXP // LANES):
    tot = zeros
    pre = zeros
    for w2 in range(N_SUB):
      row = cnts_v[w2, pl.ds(k * LANES, LANES)]
      tot = tot + row
      pre = pre + jnp.where(w2 < wid, row, zeros)
    incl = plsc.cumsum(tot)
    excl = incl - tot + carry
    base_v[pl.ds(k * LANES, LANES)] = excl + pre
    offs_v[pl.ds(k * LANES, LANES)] = excl
    carry = carry + jnp.sum(tot)
  lane = lax.iota(jnp.int32, LANES)
  offs_v[pl.ds(N_EXP, LANES)] = jnp.where(lane == 0, jnp.int32(M_TOK), 0)

  for k in range(N_EXP // LANES):
    hist_v[pl.ds(k * LANES, LANES)] = zeros
  for k in range(TOK_PER_SUB // LANES):
    idvec = ids_v[pl.ds(k * LANES, LANES)]
    occ, last = plsc.scan_count(idvec)
    prior = plsc.load_gather(hist_v, [idvec])
    basee = plsc.load_gather(base_v, [idvec])
    pos_v[0, pl.ds(k * LANES, LANES)] = basee + prior + occ - 1
    tok_v[pl.ds(k * LANES, LANES)] = tbase + k * LANES + lane
    plsc.store_scatter(hist_v, [idvec], prior + occ, mask=last)

  pltpu.sync_copy(tok_v, shared_sid.at[pos_v.at[0]])
  plsc.subcore_barrier()
  pltpu.sync_copy(shared_sid.at[pl.ds(tbase, TOK_PER_SUB)], sid_v)
  pltpu.async_copy(x_hbm.at[sid_v], rows_v, sem).wait()
  pltpu.sync_copy(rows_v, xs_hbm.at[pl.ds(tbase, TOK_PER_SUB)])

  @pl.when(wid == 0)
  def _():
    pltpu.sync_copy(offs_v, offs_hbm)


@functools.cache
def _get_sc_sort():
  return pl.kernel(
      _sc_sort_body,
      out_type=(
          jax.ShapeDtypeStruct((M_TOK, ROW_I32), jnp.int32),
          jax.ShapeDtypeStruct((OFFS_PAD,), jnp.int32),
      ),
      mesh=plsc.VectorSubcoreMesh(
          core_axis_name="c", subcore_axis_name="s", num_cores=1,
          num_subcores=N_SUB),
      compiler_params=pltpu.CompilerParams(needs_layout_passes=False),
      scratch_types=[
          pltpu.VMEM_SHARED((N_SUB, N_EXP), jnp.int32),
          pltpu.VMEM_SHARED((M_TOK,), jnp.int32),
          pltpu.VMEM((TOK_PER_SUB,), jnp.int32),
          pltpu.VMEM((N_EXP,), jnp.int32),
          pltpu.VMEM((N_SUB, N_EXP), jnp.int32),
          pltpu.VMEM((N_EXP,), jnp.int32),
          pltpu.VMEM((OFFS_PAD,), jnp.int32),
          pltpu.VMEM((1, TOK_PER_SUB), jnp.int32),
          pltpu.VMEM((TOK_PER_SUB,), jnp.int32),
          pltpu.VMEM((TOK_PER_SUB,), jnp.int32),
          pltpu.VMEM((TOK_PER_SUB, ROW_I32), jnp.int32),
          pltpu.SemaphoreType.DMA,
      ],
  )


N_TILES = M_TOK // TM


def _dec_f16(u):
  f32b = ((u & 0x8000) << 16) | (((u & 0x7FFF) << 13) + (112 << 23))
  return lax.bitcast_convert_type(f32b, jnp.float32)


HALF_D = D_IN // 2


def _tc_matmul_body(offs_ref, x_ref, w_ref, out_ref, acc_ref):
  e = pl.program_id(0)

  @pl.when(e == 0)
  def _():
    acc_ref[...] = jnp.zeros_like(acc_ref)

  off0 = offs_ref[e]
  off1 = offs_ref[e + 1]
  t0 = off0 // TM
  t1 = lax.div(off1 + TM - 1, TM)

  we = w_ref[0, pl.ds(0, HALF_D), :]
  wo = w_ref[0, pl.ds(HALF_D, HALF_D), :]

  def tile_step(k, carry):
    tt = t0 + k
    v = x_ref[tt]
    ridx = tt * TM + lax.broadcasted_iota(jnp.int32, (TM, 1), 0)
    m = (ridx >= off0) & (ridx < off1)
    xlo = jnp.where(m, _dec_f16(v & 0xFFFF), 0.).astype(jnp.bfloat16)
    xhi = jnp.where(m, _dec_f16((v >> 16) & 0xFFFF), 0.).astype(jnp.bfloat16)
    acc_ref[tt] = (acc_ref[tt]
                   + jnp.dot(xlo, we, preferred_element_type=jnp.float32)
                   + jnp.dot(xhi, wo, preferred_element_type=jnp.float32))
    return carry
  lax.fori_loop(0, t1 - t0, tile_step, 0)

  @pl.when(e == N_EXP - 1)
  def _():
    b = lax.bitcast_convert_type(acc_ref[...], jnp.int32)
    sign = (b >> 16) & 0x8000
    t = ((b & 0x7FFFFFFF) + 0x1000) >> 13
    h = jnp.where(t < (112 << 10), 0, t - (112 << 10))
    out_ref.bitcast(jnp.int16)[...] = (sign | h).astype(jnp.int16)


_tc_matmul = pl.pallas_call(
    _tc_matmul_body,
    grid_spec=pltpu.PrefetchScalarGridSpec(
        num_scalar_prefetch=1,
        grid=(N_EXP,),
        in_specs=[
            pl.BlockSpec((N_TILES, TM, ROW_I32), lambda e, s: (0, 0, 0)),
            pl.BlockSpec((1, D_IN, D_OUT), lambda e, s: (e, 0, 0)),
        ],
        out_specs=pl.BlockSpec((N_TILES, TM, D_OUT), lambda e, s: (0, 0, 0)),
        scratch_shapes=[pltpu.VMEM((N_TILES, TM, D_OUT), jnp.float32)],
    ),
    out_shape=jax.ShapeDtypeStruct((N_TILES, TM, D_OUT), jnp.float16),
    compiler_params=pltpu.CompilerParams(
        dimension_semantics=("arbitrary",)),
)


def kernel(x, expert_indices, weights):
  x_i32 = lax.bitcast_convert_type(
      jnp.stack([x[:, :HALF_D], x[:, HALF_D:]], axis=2), jnp.int32)
  xs_i32, offs = _get_sc_sort()(expert_indices, x_i32)
  w_bf = weights.astype(jnp.bfloat16)
  out = _tc_matmul(offs, xs_i32.reshape(N_TILES, TM, ROW_I32), w_bf)
  return out.reshape(M_TOK, D_OUT)
